# baseline (speedup 1.0000x reference)
import functools

import jax
import jax.numpy as jnp
from jax import lax
from jax.experimental import pallas as pl
from jax.experimental.pallas import tpu as pltpu
from jax.experimental.pallas import tpu_sc as plsc

_VC = 6400


def _sc_gather_t(feat2, idx, nv):
    nc_rows, hw = feat2.shape
    info = plsc.get_sparse_core_info()
    nw = info.num_cores * info.num_subcores
    c_total = 64
    cpt = c_total // nw
    n_cam = nc_rows // c_total
    tab_n = n_cam * hw
    tab_pad = tab_n + 16
    n_chunks = nv // _VC
    assert n_chunks * _VC == nv and _VC % 16 == 0 and cpt == 2

    mesh = plsc.VectorSubcoreMesh(core_axis_name="c", subcore_axis_name="s")

    @functools.partial(
        pl.kernel,
        mesh=mesh,
        compiler_params=pltpu.CompilerParams(
            needs_layout_passes=False, use_tc_tiling_on_sc=False),
        out_type=jax.ShapeDtypeStruct((c_total, nv), jnp.float32),
        scratch_types=[
            pltpu.VMEM((tab_pad,), jnp.float32),
            pltpu.VMEM((tab_pad,), jnp.float32),
            pltpu.VMEM((2, _VC), jnp.int32),
            pltpu.VMEM((2, _VC), jnp.float32),
            pltpu.VMEM((2, _VC), jnp.float32),
            pltpu.SemaphoreType.DMA((2,)),
            pltpu.SemaphoreType.DMA((2,)),
        ],
    )
    def k(feat_hbm, idx_hbm, out_hbm, tab0, tab1, idx_v, ob0, ob1, sem_i, sem_o):
        wid = lax.axis_index("s") * info.num_cores + lax.axis_index("c")
        c0 = wid * cpt
        c1 = c0 + 1

        for n in range(n_cam):
            pltpu.sync_copy(feat_hbm.at[n * c_total + c0], tab0.at[pl.ds(n * hw, hw)])
            pltpu.sync_copy(feat_hbm.at[n * c_total + c1], tab1.at[pl.ds(n * hw, hw)])
        zeros = jnp.zeros((16,), jnp.float32)
        tab0[pl.ds(tab_n, 16)] = zeros
        tab1[pl.ds(tab_n, 16)] = zeros

        def start_idx(j, b):
            pltpu.async_copy(idx_hbm.at[pl.ds(j * _VC, _VC)], idx_v.at[b], sem_i.at[b])

        def wait_idx(b):
            pltpu.make_async_copy(idx_hbm.at[pl.ds(0, _VC)], idx_v.at[b], sem_i.at[b]).wait()

        def start_out(j, b):
            pltpu.async_copy(ob0.at[b], out_hbm.at[c0, pl.ds(j * _VC, _VC)], sem_o.at[b])
            pltpu.async_copy(ob1.at[b], out_hbm.at[c1, pl.ds(j * _VC, _VC)], sem_o.at[b])

        def wait_out(b):
            pltpu.make_async_copy(ob0.at[b], out_hbm.at[c0, pl.ds(0, _VC)], sem_o.at[b]).wait()
            pltpu.make_async_copy(ob1.at[b], out_hbm.at[c1, pl.ds(0, _VC)], sem_o.at[b]).wait()

        def compute(b):
            def g_body(g, carry):
                iv = idx_v[b, pl.ds(g * 16, 16)]
                ob0[b, pl.ds(g * 16, 16)] = plsc.load_gather(tab0, [iv])
                ob1[b, pl.ds(g * 16, 16)] = plsc.load_gather(tab1, [iv])
                return carry
            lax.fori_loop(0, _VC // 16, g_body, 0, unroll=8)

        def body(j, b, first_round, prefetch):
            if not first_round:
                wait_out(b)
            wait_idx(b)
            compute(b)
            start_out(j, b)
            if prefetch:
                start_idx(j + 2, b)

        start_idx(0, 0)
        start_idx(1, 1)
        for j in range(2):
            body(j, j, first_round=True, prefetch=True)

        def group(g, carry):
            j0 = g * 2
            body(j0, 0, first_round=False, prefetch=True)
            body(j0 + 1, 1, first_round=False, prefetch=True)
            return carry
        lax.fori_loop(1, n_chunks // 2 - 1, group, 0)

        for j in range(n_chunks - 2, n_chunks):
            body(j, j % 2, first_round=False, prefetch=False)
        wait_out(0)
        wait_out(1)

    return k(feat2, idx)


def kernel(features_list, lut):
    feat = features_list[0]
    b, n, c, h, w = feat.shape
    nv = lut.shape[0]
    nz = 16
    nx = ny = 200

    feat2 = feat[0].reshape(n * c, h * w)

    lut32 = lut.astype(jnp.int32)
    valid = lut32[:, 0] >= 0
    flat = lut32[:, 0] * (h * w) + lut32[:, 2] * w + lut32[:, 1]
    idx = jnp.where(valid, flat, n * h * w).astype(jnp.int32)

    out_t = _sc_gather_t(feat2, idx, nv)
    return (out_t.reshape(1, c, nx, ny, nz),)


# speedup vs baseline: 3.1629x; 3.1629x over previous
import functools

import jax
import jax.numpy as jnp
from jax import lax
from jax.experimental import pallas as pl
from jax.experimental.pallas import tpu as pltpu
from jax.experimental.pallas import tpu_sc as plsc

_XPC = 2


def _sc_gather_t(feat2, idx, nx, ny, nz):
    nc_rows, hw = feat2.shape
    info = plsc.get_sparse_core_info()
    nw = info.num_cores * info.num_subcores
    c_total = 64
    cpt = c_total // nw
    n_cam = nc_rows // c_total
    tab_n = n_cam * hw
    tab_pad = tab_n + 16
    vc = _XPC * ny * nz
    n_chunks = nx // _XPC
    assert n_chunks * _XPC == nx and ny * nz % 16 == 0 and cpt == 2

    mesh = plsc.VectorSubcoreMesh(core_axis_name="c", subcore_axis_name="s")

    @functools.partial(
        pl.kernel,
        mesh=mesh,
        compiler_params=pltpu.CompilerParams(
            needs_layout_passes=False, use_tc_tiling_on_sc=False),
        out_type=jax.ShapeDtypeStruct((1, c_total, nx, ny, nz), jnp.float32),
        scratch_types=[
            pltpu.VMEM((tab_pad,), jnp.float32),
            pltpu.VMEM((tab_pad,), jnp.float32),
            pltpu.VMEM((2, vc), jnp.int32),
            pltpu.VMEM((2, _XPC, ny, nz), jnp.float32),
            pltpu.VMEM((2, _XPC, ny, nz), jnp.float32),
            pltpu.SemaphoreType.DMA((2,)),
            pltpu.SemaphoreType.DMA((2,)),
        ],
    )
    def k(feat_hbm, idx_hbm, out_hbm, tab0, tab1, idx_v, ob0, ob1, sem_i, sem_o):
        wid = lax.axis_index("s") * info.num_cores + lax.axis_index("c")
        c0 = wid * cpt
        c1 = c0 + 1

        for n in range(n_cam):
            pltpu.sync_copy(feat_hbm.at[n * c_total + c0], tab0.at[pl.ds(n * hw, hw)])
            pltpu.sync_copy(feat_hbm.at[n * c_total + c1], tab1.at[pl.ds(n * hw, hw)])
        zeros = jnp.zeros((16,), jnp.float32)
        tab0[pl.ds(tab_n, 16)] = zeros
        tab1[pl.ds(tab_n, 16)] = zeros

        def start_idx(j, b):
            pltpu.async_copy(idx_hbm.at[pl.ds(j * vc, vc)], idx_v.at[b], sem_i.at[b])

        def wait_idx(b):
            pltpu.make_async_copy(idx_hbm.at[pl.ds(0, vc)], idx_v.at[b], sem_i.at[b]).wait()

        def start_out(j, b):
            pltpu.async_copy(ob0.at[b], out_hbm.at[0, c0, pl.ds(j * _XPC, _XPC)], sem_o.at[b])
            pltpu.async_copy(ob1.at[b], out_hbm.at[0, c1, pl.ds(j * _XPC, _XPC)], sem_o.at[b])

        def wait_out(b):
            pltpu.make_async_copy(ob0.at[b], out_hbm.at[0, c0, pl.ds(0, _XPC)], sem_o.at[b]).wait()
            pltpu.make_async_copy(ob1.at[b], out_hbm.at[0, c1, pl.ds(0, _XPC)], sem_o.at[b]).wait()

        def compute(b):
            for a in range(_XPC):
                @plsc.parallel_loop(0, ny, unroll=8)
                def y_body(y):
                    iv = idx_v[b, pl.ds(a * ny * nz + y * nz, 16)]
                    ob0[b, a, y, :] = plsc.load_gather(tab0, [iv])
                    ob1[b, a, y, :] = plsc.load_gather(tab1, [iv])

        def body(j, b, first_round, prefetch):
            if not first_round:
                wait_out(b)
            wait_idx(b)
            compute(b)
            start_out(j, b)
            if prefetch:
                start_idx(j + 2, b)

        start_idx(0, 0)
        start_idx(1, 1)
        for j in range(2):
            body(j, j, first_round=True, prefetch=True)

        def group(g, carry):
            j0 = g * 2
            body(j0, 0, first_round=False, prefetch=True)
            body(j0 + 1, 1, first_round=False, prefetch=True)
            return carry
        lax.fori_loop(1, n_chunks // 2 - 1, group, 0)

        for j in range(n_chunks - 2, n_chunks):
            body(j, j % 2, first_round=False, prefetch=False)
        wait_out(0)
        wait_out(1)

    return k(feat2, idx)


def kernel(features_list, lut):
    feat = features_list[0]
    b, n, c, h, w = feat.shape
    nz = 16
    nx = ny = 200

    feat2 = feat[0].reshape(n * c, h * w)

    lut32 = lut.astype(jnp.int32)
    valid = lut32[:, 0] >= 0
    flat = lut32[:, 0] * (h * w) + lut32[:, 2] * w + lut32[:, 1]
    idx = jnp.where(valid, flat, n * h * w).astype(jnp.int32)

    return (_sc_gather_t(feat2, idx, nx, ny, nz),)


# speedup vs baseline: 7.0722x; 2.2360x over previous
import functools

import jax
import jax.numpy as jnp
from jax import lax
from jax.experimental import pallas as pl
from jax.experimental.pallas import tpu as pltpu
from jax.experimental.pallas import tpu_sc as plsc

_XPC = 2


def _sc_gather_t(feat2, idx, nx, ny, nz):
    nc_rows, hw = feat2.shape
    info = plsc.get_sparse_core_info()
    nw = info.num_cores * info.num_subcores
    c_total = 64
    cpt = c_total // nw
    n_cam = nc_rows // c_total
    tab_n = n_cam * hw
    tab_pad = tab_n + 16
    vc = _XPC * ny * nz
    n_chunks = nx // _XPC
    assert n_chunks * _XPC == nx and cpt == 2
    y0s = [min(g * 16, ny - 16) for g in range(-(-ny // 16))]

    mesh = plsc.VectorSubcoreMesh(core_axis_name="c", subcore_axis_name="s")

    @functools.partial(
        pl.kernel,
        mesh=mesh,
        compiler_params=pltpu.CompilerParams(
            needs_layout_passes=False, use_tc_tiling_on_sc=False),
        out_type=jax.ShapeDtypeStruct((1, c_total, nx, nz, ny), jnp.float32),
        scratch_types=[
            pltpu.VMEM((tab_pad,), jnp.float32),
            pltpu.VMEM((tab_pad,), jnp.float32),
            pltpu.VMEM((2 * vc,), jnp.int32),
            pltpu.VMEM((2, _XPC, nz, ny), jnp.float32),
            pltpu.VMEM((2, _XPC, nz, ny), jnp.float32),
            pltpu.SemaphoreType.DMA((2,)),
            pltpu.SemaphoreType.DMA((2,)),
        ],
    )
    def k(feat_hbm, idx_hbm, out_hbm, tab0, tab1, idx_v, ob0, ob1, sem_i, sem_o):
        wid = lax.axis_index("s") * info.num_cores + lax.axis_index("c")
        c0 = wid * cpt
        c1 = c0 + 1

        for n in range(n_cam):
            pltpu.sync_copy(feat_hbm.at[n * c_total + c0], tab0.at[pl.ds(n * hw, hw)])
            pltpu.sync_copy(feat_hbm.at[n * c_total + c1], tab1.at[pl.ds(n * hw, hw)])
        zeros = jnp.zeros((16,), jnp.float32)
        tab0[pl.ds(tab_n, 16)] = zeros
        tab1[pl.ds(tab_n, 16)] = zeros

        def start_idx(j, b):
            pltpu.async_copy(idx_hbm.at[pl.ds(j * vc, vc)],
                             idx_v.at[pl.ds(b * vc, vc)], sem_i.at[b])

        def wait_idx(b):
            pltpu.make_async_copy(idx_hbm.at[pl.ds(0, vc)],
                                  idx_v.at[pl.ds(b * vc, vc)], sem_i.at[b]).wait()

        def start_out(j, b):
            pltpu.async_copy(ob0.at[b], out_hbm.at[0, c0, pl.ds(j * _XPC, _XPC)], sem_o.at[b])
            pltpu.async_copy(ob1.at[b], out_hbm.at[0, c1, pl.ds(j * _XPC, _XPC)], sem_o.at[b])

        def wait_out(b):
            pltpu.make_async_copy(ob0.at[b], out_hbm.at[0, c0, pl.ds(0, _XPC)], sem_o.at[b]).wait()
            pltpu.make_async_copy(ob1.at[b], out_hbm.at[0, c1, pl.ds(0, _XPC)], sem_o.at[b]).wait()

        stride16 = lax.iota(jnp.int32, 16) * nz

        def compute(b):
            for a in range(_XPC):
                plane = b * vc + a * ny * nz

                @plsc.parallel_loop(0, nz, unroll=2)
                def z_body(z):
                    for y0 in y0s:
                        iv = plsc.load_gather(idx_v, [stride16 + (plane + y0 * nz + z)])
                        ob0[b, a, z, pl.ds(y0, 16)] = plsc.load_gather(tab0, [iv])
                        ob1[b, a, z, pl.ds(y0, 16)] = plsc.load_gather(tab1, [iv])

        def body(j, b, first_round, prefetch):
            if not first_round:
                wait_out(b)
            wait_idx(b)
            compute(b)
            start_out(j, b)
            if prefetch:
                start_idx(j + 2, b)

        start_idx(0, 0)
        start_idx(1, 1)
        for j in range(2):
            body(j, j, first_round=True, prefetch=True)

        def group(g, carry):
            j0 = g * 2
            body(j0, 0, first_round=False, prefetch=True)
            body(j0 + 1, 1, first_round=False, prefetch=True)
            return carry
        lax.fori_loop(1, n_chunks // 2 - 1, group, 0)

        for j in range(n_chunks - 2, n_chunks):
            body(j, j % 2, first_round=False, prefetch=False)
        wait_out(0)
        wait_out(1)

    return k(feat2, idx)


def kernel(features_list, lut):
    feat = features_list[0]
    b, n, c, h, w = feat.shape
    nz = 16
    nx = ny = 200

    feat2 = feat[0].reshape(n * c, h * w)

    lut32 = lut.astype(jnp.int32)
    valid = lut32[:, 0] >= 0
    flat = lut32[:, 0] * (h * w) + lut32[:, 2] * w + lut32[:, 1]
    idx = jnp.where(valid, flat, n * h * w).astype(jnp.int32)

    out_zy = _sc_gather_t(feat2, idx, nx, ny, nz)
    return (jnp.swapaxes(out_zy, 3, 4),)


# speedup vs baseline: 7.1816x; 1.0155x over previous
import functools

import jax
import jax.numpy as jnp
from jax import lax
from jax.experimental import pallas as pl
from jax.experimental.pallas import tpu as pltpu
from jax.experimental.pallas import tpu_sc as plsc

_XPC = 2


def _sc_gather_t(feat2, idx, nx, ny, nz):
    nc_rows, hw = feat2.shape
    info = plsc.get_sparse_core_info()
    nw = info.num_cores * info.num_subcores
    c_total = 64
    cpt = c_total // nw
    n_cam = nc_rows // c_total
    tab_n = n_cam * hw
    tab_pad = tab_n + 16
    vc = _XPC * ny * nz
    n_chunks = nx // _XPC
    assert n_chunks * _XPC == nx and cpt == 2
    y0s = [min(g * 16, ny - 16) for g in range(-(-ny // 16))]

    mesh = plsc.VectorSubcoreMesh(core_axis_name="c", subcore_axis_name="s")

    @functools.partial(
        pl.kernel,
        mesh=mesh,
        compiler_params=pltpu.CompilerParams(
            needs_layout_passes=False, use_tc_tiling_on_sc=False),
        out_type=jax.ShapeDtypeStruct((1, c_total, nx, nz, ny), jnp.float32),
        scratch_types=[
            pltpu.VMEM((tab_pad,), jnp.float32),
            pltpu.VMEM((tab_pad,), jnp.float32),
            pltpu.VMEM((2 * vc,), jnp.int32),
            pltpu.VMEM((2, _XPC, nz, ny), jnp.float32),
            pltpu.VMEM((2, _XPC, nz, ny), jnp.float32),
            pltpu.SemaphoreType.DMA((2,)),
            pltpu.SemaphoreType.DMA((2,)),
        ],
    )
    def k(feat_hbm, idx_hbm, out_hbm, tab0, tab1, idx_v, ob0, ob1, sem_i, sem_o):
        wid = lax.axis_index("s") * info.num_cores + lax.axis_index("c")
        c0 = wid * cpt
        c1 = c0 + 1

        for n in range(n_cam):
            pltpu.sync_copy(feat_hbm.at[n * c_total + c0], tab0.at[pl.ds(n * hw, hw)])
            pltpu.sync_copy(feat_hbm.at[n * c_total + c1], tab1.at[pl.ds(n * hw, hw)])
        zeros = jnp.zeros((16,), jnp.float32)
        tab0[pl.ds(tab_n, 16)] = zeros
        tab1[pl.ds(tab_n, 16)] = zeros

        def start_idx(j, b):
            pltpu.async_copy(idx_hbm.at[pl.ds(j * vc, vc)],
                             idx_v.at[pl.ds(b * vc, vc)], sem_i.at[b])

        def wait_idx(b):
            pltpu.make_async_copy(idx_hbm.at[pl.ds(0, vc)],
                                  idx_v.at[pl.ds(b * vc, vc)], sem_i.at[b]).wait()

        def start_out(j, b):
            pltpu.async_copy(ob0.at[b], out_hbm.at[0, c0, pl.ds(j * _XPC, _XPC)], sem_o.at[b])
            pltpu.async_copy(ob1.at[b], out_hbm.at[0, c1, pl.ds(j * _XPC, _XPC)], sem_o.at[b])

        def wait_out(b):
            pltpu.make_async_copy(ob0.at[b], out_hbm.at[0, c0, pl.ds(0, _XPC)], sem_o.at[b]).wait()
            pltpu.make_async_copy(ob1.at[b], out_hbm.at[0, c1, pl.ds(0, _XPC)], sem_o.at[b]).wait()

        def compute(b):
            for a in range(_XPC):
                plane = b * vc + a * ny * nz

                @plsc.parallel_loop(0, nz, unroll=2)
                def z_body(z):
                    for y0 in y0s:
                        iv = idx_v[pl.ds(plane + z * ny + y0, 16)]
                        ob0[b, a, z, pl.ds(y0, 16)] = plsc.load_gather(tab0, [iv])
                        ob1[b, a, z, pl.ds(y0, 16)] = plsc.load_gather(tab1, [iv])

        def body(j, b, first_round, prefetch):
            if not first_round:
                wait_out(b)
            wait_idx(b)
            compute(b)
            start_out(j, b)
            if prefetch:
                start_idx(j + 2, b)

        start_idx(0, 0)
        start_idx(1, 1)
        for j in range(2):
            body(j, j, first_round=True, prefetch=True)

        def group(g, carry):
            j0 = g * 2
            body(j0, 0, first_round=False, prefetch=True)
            body(j0 + 1, 1, first_round=False, prefetch=True)
            return carry
        lax.fori_loop(1, n_chunks // 2 - 1, group, 0)

        for j in range(n_chunks - 2, n_chunks):
            body(j, j % 2, first_round=False, prefetch=False)
        wait_out(0)
        wait_out(1)

    return k(feat2, idx)


def kernel(features_list, lut):
    feat = features_list[0]
    b, n, c, h, w = feat.shape
    nz = 16
    nx = ny = 200

    feat2 = feat[0].reshape(n * c, h * w)

    lut32 = lut.astype(jnp.int32)
    valid = lut32[:, 0] >= 0
    flat = lut32[:, 0] * (h * w) + lut32[:, 2] * w + lut32[:, 1]
    idx = jnp.where(valid, flat, n * h * w).astype(jnp.int32)
    idx_zy = idx.reshape(nx, ny, nz).transpose(0, 2, 1).reshape(-1)

    out_zy = _sc_gather_t(feat2, idx_zy, nx, ny, nz)
    return (jnp.swapaxes(out_zy, 3, 4),)
